# baseline (speedup 1.0000x reference)
import jax
import jax.numpy as jnp
from jax.experimental import pallas as pl
from jax.experimental.pallas import tpu as pltpu

V = 3
N = 2000
D_IN = 256
H1 = 128
H2 = 64
K = 10
TS = 400
NT = N // TS
ND = 4

_F32 = jnp.float32


def _softmax_w(fw_ref):
    e0 = jnp.exp(fw_ref[0, 0])
    e1 = jnp.exp(fw_ref[0, 1])
    e2 = jnp.exp(fw_ref[0, 2])
    s = e0 + e1 + e2
    return e0 / s, e1 / s, e2 / s


def _wv(fw_ref, v):
    w0, w1, w2 = _softmax_w(fw_ref)
    return jnp.where(v == 0, w0, jnp.where(v == 1, w1, w2))


def _dot(x, y, dims=(((1,), (0,)), ((), ()))):
    return jax.lax.dot_general(x, y, dims, preferred_element_type=_F32)


def _gcn_kernel(x_ref, a_ref, w1_ref, b1_ref, w2_ref, b2_ref,
                fgw_ref, fw_ref,
                adjbar_ref, h_ref, g_ref, m_ref, cpr_ref, gsum_ref,
                s_scr, sems):
    v = pl.program_id(0)
    a = a_ref[0]
    p1 = _dot(x_ref[0], w1_ref[0])
    h1 = jax.nn.relu(_dot(a, p1) + b1_ref[0])
    p2 = _dot(h1, w2_ref[0])
    h = jax.nn.relu(_dot(a, p2) + b2_ref[0])
    h_ref[0] = h
    g = _dot(h, fgw_ref[...])
    g_ref[0] = g
    wv = _wv(fw_ref, v)

    @pl.when(v == 0)
    def _():
        cpr_ref[...] = wv * h
        gsum_ref[...] = g

    @pl.when(v > 0)
    def _():
        cpr_ref[...] += wv * h
        gsum_ref[...] += g

    copies = [None] * NT
    for i in range(NT):
        buf = i % ND
        if i >= ND:
            copies[i - ND].wait()
        else:
            @pl.when(v > 0)
            def _(buf=buf, i=i):
                pltpu.make_async_copy(
                    s_scr.at[buf],
                    adjbar_ref.at[jnp.maximum(v - 1, 0),
                                  pl.ds((NT - ND + i) * TS, TS), :],
                    sems.at[buf]).wait()
        s = jax.nn.sigmoid(_dot(h[i * TS:(i + 1) * TS], h,
                                (((1,), (1,)), ((), ()))))
        s_scr[buf] = s
        cp = pltpu.make_async_copy(
            s_scr.at[buf],
            adjbar_ref.at[v, pl.ds(i * TS, TS), :],
            sems.at[buf])
        cp.start()
        copies[i] = cp
        m_ref[0, i * TS:(i + 1) * TS, :] = _dot(s, g)

    @pl.when(v == V - 1)
    def _():
        for i in range(max(NT - ND, 0), NT):
            copies[i].wait()


def _combine_kernel(m_ref, g_ref, gsum_ref, fgb_ref, fw_ref, cen_ref,
                    h_ref, dw1_ref, db1_ref, dw2_ref, db2_ref,
                    comb_ref, q_ref, xb_ref):
    for i in range(V):
        xb = jax.nn.relu(_dot(h_ref[i], dw1_ref[i]) + db1_ref[i])
        xb_ref[i] = jax.nn.relu(_dot(xb, dw2_ref[i]) + db2_ref[i])
    w0, w1, w2 = _softmax_w(fw_ref)
    gsum = gsum_ref[...] + fgb_ref[...]
    c = (w0 * jax.nn.relu(m_ref[0] + gsum - g_ref[0])
         + w1 * jax.nn.relu(m_ref[1] + gsum - g_ref[1])
         + w2 * jax.nn.relu(m_ref[2] + gsum - g_ref[2]))
    comb_ref[...] = c
    cen = cen_ref[...]
    cs = jnp.sum(c * c, axis=1, keepdims=True)
    cc = jnp.sum(cen * cen, axis=1, keepdims=True).reshape(1, K)
    dist = cs + cc - 2.0 * jax.lax.dot_general(
        c, cen, (((1,), (1,)), ((), ())), preferred_element_type=_F32)
    q = 1.0 / (1.0 + dist)
    q_ref[...] = q / jnp.sum(q, axis=1, keepdims=True)


def kernel(feats, adjs, pm_W1, pm_b1, pm_W2, pm_b2, de_W1, de_b1, de_W2,
           de_b2, fg_W, fg_b, fusion_w, centers):
    f32 = _F32
    pm_b1r = pm_b1.reshape(V, 1, H1)
    pm_b2r = pm_b2.reshape(V, 1, H2)
    de_b1r = de_b1.reshape(V, 1, H1)
    de_b2r = de_b2.reshape(V, 1, D_IN)
    fg_br = fg_b.reshape(1, H2)
    fw = jnp.zeros((8, 128), f32).at[0, :V].set(fusion_w)

    adjbar, h, g, m, combined_pr, gsum = pl.pallas_call(
        _gcn_kernel,
        grid=(V,),
        in_specs=[
            pl.BlockSpec((1, N, D_IN), lambda v: (v, 0, 0)),
            pl.BlockSpec((1, N, N), lambda v: (v, 0, 0)),
            pl.BlockSpec((1, D_IN, H1), lambda v: (v, 0, 0)),
            pl.BlockSpec((1, 1, H1), lambda v: (v, 0, 0)),
            pl.BlockSpec((1, H1, H2), lambda v: (v, 0, 0)),
            pl.BlockSpec((1, 1, H2), lambda v: (v, 0, 0)),
            pl.BlockSpec((H2, H2), lambda v: (0, 0)),
            pl.BlockSpec((8, 128), lambda v: (0, 0)),
        ],
        out_specs=[
            pl.BlockSpec(memory_space=pl.ANY),
            pl.BlockSpec((1, N, H2), lambda v: (v, 0, 0)),
            pl.BlockSpec((1, N, H2), lambda v: (v, 0, 0)),
            pl.BlockSpec((1, N, H2), lambda v: (v, 0, 0)),
            pl.BlockSpec((N, H2), lambda v: (0, 0)),
            pl.BlockSpec((N, H2), lambda v: (0, 0)),
        ],
        out_shape=[
            jax.ShapeDtypeStruct((V, N, N), f32),
            jax.ShapeDtypeStruct((V, N, H2), f32),
            jax.ShapeDtypeStruct((V, N, H2), f32),
            jax.ShapeDtypeStruct((V, N, H2), f32),
            jax.ShapeDtypeStruct((N, H2), f32),
            jax.ShapeDtypeStruct((N, H2), f32),
        ],
        scratch_shapes=[
            pltpu.VMEM((ND, TS, N), f32),
            pltpu.SemaphoreType.DMA((ND,)),
        ],
    )(feats, adjs, pm_W1, pm_b1r, pm_W2, pm_b2r, fg_W, fw)

    combined, q, xbar = pl.pallas_call(
        _combine_kernel,
        grid=(1,),
        in_specs=[
            pl.BlockSpec((V, N, H2), lambda i: (0, 0, 0)),
            pl.BlockSpec((V, N, H2), lambda i: (0, 0, 0)),
            pl.BlockSpec((N, H2), lambda i: (0, 0)),
            pl.BlockSpec((1, H2), lambda i: (0, 0)),
            pl.BlockSpec((8, 128), lambda i: (0, 0)),
            pl.BlockSpec((K, H2), lambda i: (0, 0)),
            pl.BlockSpec((V, N, H2), lambda i: (0, 0, 0)),
            pl.BlockSpec((V, H2, H1), lambda i: (0, 0, 0)),
            pl.BlockSpec((V, 1, H1), lambda i: (0, 0, 0)),
            pl.BlockSpec((V, H1, D_IN), lambda i: (0, 0, 0)),
            pl.BlockSpec((V, 1, D_IN), lambda i: (0, 0, 0)),
        ],
        out_specs=[
            pl.BlockSpec((N, H2), lambda i: (0, 0)),
            pl.BlockSpec((N, K), lambda i: (0, 0)),
            pl.BlockSpec((V, N, D_IN), lambda i: (0, 0, 0)),
        ],
        out_shape=[
            jax.ShapeDtypeStruct((N, H2), f32),
            jax.ShapeDtypeStruct((N, K), f32),
            jax.ShapeDtypeStruct((V, N, D_IN), f32),
        ],
    )(m, g, gsum, fg_br, fw, centers, h, de_W1, de_b1r, de_W2, de_b2r)

    return (combined, combined_pr, q, xbar, adjbar)


# speedup vs baseline: 1.0186x; 1.0186x over previous
import jax
import jax.numpy as jnp
from jax.experimental import pallas as pl
from jax.experimental.pallas import tpu as pltpu

V = 3
N = 2000
D_IN = 256
H1 = 128
H2 = 64
K = 10
TS = 400
NT = N // TS
ND = 2

_F32 = jnp.float32


def _softmax_w(fw_ref):
    e0 = jnp.exp(fw_ref[0, 0])
    e1 = jnp.exp(fw_ref[0, 1])
    e2 = jnp.exp(fw_ref[0, 2])
    s = e0 + e1 + e2
    return e0 / s, e1 / s, e2 / s


def _wv(fw_ref, v):
    w0, w1, w2 = _softmax_w(fw_ref)
    return jnp.where(v == 0, w0, jnp.where(v == 1, w1, w2))


def _dot(x, y, dims=(((1,), (0,)), ((), ()))):
    return jax.lax.dot_general(x, y, dims, preferred_element_type=_F32)


def _gcn_kernel(x_ref, a_ref, w1_ref, b1_ref, w2_ref, b2_ref,
                dw1_ref, db1_ref, dw2_ref, db2_ref,
                fgw_ref, fw_ref, fgb_ref, cen_ref,
                adjbar_ref, xb_ref, cpr_ref, comb_ref, q_ref,
                s_scr, sems, g_scr, m_scr):
    v = pl.program_id(0)
    a = a_ref[0]
    p1 = _dot(x_ref[0], w1_ref[0])
    h1 = jax.nn.relu(_dot(a, p1) + b1_ref[0])
    p2 = _dot(h1, w2_ref[0])
    h = jax.nn.relu(_dot(a, p2) + b2_ref[0])
    for r in range(2):
        rows = slice(r * (N // 2), (r + 1) * (N // 2))
        xb = jax.nn.relu(_dot(h[rows], dw1_ref[0]) + db1_ref[0])
        xb_ref[0, rows, :] = jax.nn.relu(_dot(xb, dw2_ref[0]) + db2_ref[0])
    g = _dot(h, fgw_ref[...])
    g_scr[v] = g
    wv = _wv(fw_ref, v)

    @pl.when(v == 0)
    def _():
        cpr_ref[...] = wv * h

    @pl.when(v > 0)
    def _():
        cpr_ref[...] += wv * h

    copies = [None] * NT
    for i in range(NT):
        buf = i % ND
        if i >= ND:
            copies[i - ND].wait()
        else:
            @pl.when(v > 0)
            def _(buf=buf, i=i):
                pltpu.make_async_copy(
                    s_scr.at[buf],
                    adjbar_ref.at[jnp.maximum(v - 1, 0),
                                  pl.ds((NT - ND + i) * TS, TS), :],
                    sems.at[buf]).wait()
        s = jax.nn.sigmoid(_dot(h[i * TS:(i + 1) * TS], h,
                                (((1,), (1,)), ((), ()))))
        s_scr[buf] = s
        cp = pltpu.make_async_copy(
            s_scr.at[buf],
            adjbar_ref.at[v, pl.ds(i * TS, TS), :],
            sems.at[buf])
        cp.start()
        copies[i] = cp
        m_scr[v, i * TS:(i + 1) * TS, :] = _dot(s, g)

    @pl.when(v == V - 1)
    def _():
        for i in range(max(NT - ND, 0), NT):
            copies[i].wait()
        w0, w1, w2 = _softmax_w(fw_ref)
        cen = cen_ref[...]
        cc = jnp.sum(cen * cen, axis=1, keepdims=True).reshape(1, K)
        CH = N // 4
        for r in range(4):
            rows = slice(r * CH, (r + 1) * CH)
            g0, g1, g2 = g_scr[0, rows], g_scr[1, rows], g_scr[2, rows]
            gsum = g0 + g1 + g2 + fgb_ref[...]
            c = (w0 * jax.nn.relu(m_scr[0, rows] + gsum - g0)
                 + w1 * jax.nn.relu(m_scr[1, rows] + gsum - g1)
                 + w2 * jax.nn.relu(m_scr[2, rows] + gsum - g2))
            comb_ref[rows, :] = c
            cs = jnp.sum(c * c, axis=1, keepdims=True)
            dist = cs + cc - 2.0 * jax.lax.dot_general(
                c, cen, (((1,), (1,)), ((), ())), preferred_element_type=_F32)
            q = 1.0 / (1.0 + dist)
            q_ref[rows, :] = q / jnp.sum(q, axis=1, keepdims=True)


def kernel(feats, adjs, pm_W1, pm_b1, pm_W2, pm_b2, de_W1, de_b1, de_W2,
           de_b2, fg_W, fg_b, fusion_w, centers):
    f32 = _F32
    pm_b1r = pm_b1.reshape(V, 1, H1)
    pm_b2r = pm_b2.reshape(V, 1, H2)
    de_b1r = de_b1.reshape(V, 1, H1)
    de_b2r = de_b2.reshape(V, 1, D_IN)
    fg_br = fg_b.reshape(1, H2)
    fw = jnp.zeros((8, 128), f32).at[0, :V].set(fusion_w)

    adjbar, xbar, combined_pr, combined, q = pl.pallas_call(
        _gcn_kernel,
        grid=(V,),
        in_specs=[
            pl.BlockSpec((1, N, D_IN), lambda v: (v, 0, 0)),
            pl.BlockSpec((1, N, N), lambda v: (v, 0, 0)),
            pl.BlockSpec((1, D_IN, H1), lambda v: (v, 0, 0)),
            pl.BlockSpec((1, 1, H1), lambda v: (v, 0, 0)),
            pl.BlockSpec((1, H1, H2), lambda v: (v, 0, 0)),
            pl.BlockSpec((1, 1, H2), lambda v: (v, 0, 0)),
            pl.BlockSpec((1, H2, H1), lambda v: (v, 0, 0)),
            pl.BlockSpec((1, 1, H1), lambda v: (v, 0, 0)),
            pl.BlockSpec((1, H1, D_IN), lambda v: (v, 0, 0)),
            pl.BlockSpec((1, 1, D_IN), lambda v: (v, 0, 0)),
            pl.BlockSpec((H2, H2), lambda v: (0, 0)),
            pl.BlockSpec((8, 128), lambda v: (0, 0)),
            pl.BlockSpec((1, H2), lambda v: (0, 0)),
            pl.BlockSpec((K, H2), lambda v: (0, 0)),
        ],
        out_specs=[
            pl.BlockSpec(memory_space=pl.ANY),
            pl.BlockSpec((1, N, D_IN), lambda v: (v, 0, 0)),
            pl.BlockSpec((N, H2), lambda v: (0, 0)),
            pl.BlockSpec((N, H2), lambda v: (0, 0)),
            pl.BlockSpec((N, K), lambda v: (0, 0)),
        ],
        out_shape=[
            jax.ShapeDtypeStruct((V, N, N), f32),
            jax.ShapeDtypeStruct((V, N, D_IN), f32),
            jax.ShapeDtypeStruct((N, H2), f32),
            jax.ShapeDtypeStruct((N, H2), f32),
            jax.ShapeDtypeStruct((N, K), f32),
        ],
        scratch_shapes=[
            pltpu.VMEM((ND, TS, N), f32),
            pltpu.SemaphoreType.DMA((ND,)),
            pltpu.VMEM((V, N, H2), f32),
            pltpu.VMEM((V, N, H2), f32),
        ],
    )(feats, adjs, pm_W1, pm_b1r, pm_W2, pm_b2r,
      de_W1, de_b1r, de_W2, de_b2r, fg_W, fw, fg_br, centers)

    return (combined, combined_pr, q, xbar, adjbar)


# speedup vs baseline: 1.0275x; 1.0088x over previous
import jax
import jax.numpy as jnp
from jax.experimental import pallas as pl
from jax.experimental.pallas import tpu as pltpu

V = 3
N = 2000
D_IN = 256
H1 = 128
H2 = 64
K = 10
TS = 400
NT = N // TS
ND = 3

_F32 = jnp.float32


def _softmax_w(fw_ref):
    e0 = jnp.exp(fw_ref[0, 0])
    e1 = jnp.exp(fw_ref[0, 1])
    e2 = jnp.exp(fw_ref[0, 2])
    s = e0 + e1 + e2
    return e0 / s, e1 / s, e2 / s


def _wv(fw_ref, v):
    w0, w1, w2 = _softmax_w(fw_ref)
    return jnp.where(v == 0, w0, jnp.where(v == 1, w1, w2))


def _dot(x, y, dims=(((1,), (0,)), ((), ()))):
    return jax.lax.dot_general(x, y, dims, preferred_element_type=_F32)


def _gcn_kernel(x_ref, a_ref, w1_ref, b1_ref, w2_ref, b2_ref,
                dw1_ref, db1_ref, dw2_ref, db2_ref,
                fgw_ref, fw_ref,
                adjbar_ref, xb_ref, g_ref, m_ref, cpr_ref, gsum_ref,
                s_scr, sems):
    v = pl.program_id(0)
    a = a_ref[0]
    p1 = _dot(x_ref[0], w1_ref[0])
    h1 = jax.nn.relu(_dot(a, p1) + b1_ref[0])
    p2 = _dot(h1, w2_ref[0])
    h = jax.nn.relu(_dot(a, p2) + b2_ref[0])
    xb = jax.nn.relu(_dot(h, dw1_ref[0]) + db1_ref[0])
    xb_ref[0] = jax.nn.relu(_dot(xb, dw2_ref[0]) + db2_ref[0])
    g = _dot(h, fgw_ref[...])
    g_ref[0] = g
    wv = _wv(fw_ref, v)

    @pl.when(v == 0)
    def _():
        cpr_ref[...] = wv * h
        gsum_ref[...] = g

    @pl.when(v > 0)
    def _():
        cpr_ref[...] += wv * h
        gsum_ref[...] += g

    copies = [None] * NT
    for i in range(NT):
        buf = i % ND
        if i >= ND:
            copies[i - ND].wait()
        else:
            @pl.when(v > 0)
            def _(buf=buf, i=i):
                pltpu.make_async_copy(
                    s_scr.at[buf],
                    adjbar_ref.at[jnp.maximum(v - 1, 0),
                                  pl.ds((NT - ND + i) * TS, TS), :],
                    sems.at[buf]).wait()
        s = jax.nn.sigmoid(_dot(h[i * TS:(i + 1) * TS], h,
                                (((1,), (1,)), ((), ()))))
        s_scr[buf] = s
        cp = pltpu.make_async_copy(
            s_scr.at[buf],
            adjbar_ref.at[v, pl.ds(i * TS, TS), :],
            sems.at[buf])
        cp.start()
        copies[i] = cp
        m_ref[0, i * TS:(i + 1) * TS, :] = _dot(s, g)

    @pl.when(v == V - 1)
    def _():
        for i in range(max(NT - ND, 0), NT):
            copies[i].wait()


def _combine_kernel(m_ref, g_ref, gsum_ref, fgb_ref, fw_ref, cen_ref,
                    comb_ref, q_ref):
    w0, w1, w2 = _softmax_w(fw_ref)
    gsum = gsum_ref[...] + fgb_ref[...]
    c = (w0 * jax.nn.relu(m_ref[0] + gsum - g_ref[0])
         + w1 * jax.nn.relu(m_ref[1] + gsum - g_ref[1])
         + w2 * jax.nn.relu(m_ref[2] + gsum - g_ref[2]))
    comb_ref[...] = c
    cen = cen_ref[...]
    cs = jnp.sum(c * c, axis=1, keepdims=True)
    cc = jnp.sum(cen * cen, axis=1, keepdims=True).reshape(1, K)
    dist = cs + cc - 2.0 * jax.lax.dot_general(
        c, cen, (((1,), (1,)), ((), ())), preferred_element_type=_F32)
    q = 1.0 / (1.0 + dist)
    q_ref[...] = q / jnp.sum(q, axis=1, keepdims=True)


def kernel(feats, adjs, pm_W1, pm_b1, pm_W2, pm_b2, de_W1, de_b1, de_W2,
           de_b2, fg_W, fg_b, fusion_w, centers):
    f32 = _F32
    pm_b1r = pm_b1.reshape(V, 1, H1)
    pm_b2r = pm_b2.reshape(V, 1, H2)
    de_b1r = de_b1.reshape(V, 1, H1)
    de_b2r = de_b2.reshape(V, 1, D_IN)
    fg_br = fg_b.reshape(1, H2)
    fw = jnp.zeros((8, 128), f32).at[0, :V].set(fusion_w)

    adjbar, xbar, g, m, combined_pr, gsum = pl.pallas_call(
        _gcn_kernel,
        grid=(V,),
        in_specs=[
            pl.BlockSpec((1, N, D_IN), lambda v: (v, 0, 0)),
            pl.BlockSpec((1, N, N), lambda v: (v, 0, 0)),
            pl.BlockSpec((1, D_IN, H1), lambda v: (v, 0, 0)),
            pl.BlockSpec((1, 1, H1), lambda v: (v, 0, 0)),
            pl.BlockSpec((1, H1, H2), lambda v: (v, 0, 0)),
            pl.BlockSpec((1, 1, H2), lambda v: (v, 0, 0)),
            pl.BlockSpec((1, H2, H1), lambda v: (v, 0, 0)),
            pl.BlockSpec((1, 1, H1), lambda v: (v, 0, 0)),
            pl.BlockSpec((1, H1, D_IN), lambda v: (v, 0, 0)),
            pl.BlockSpec((1, 1, D_IN), lambda v: (v, 0, 0)),
            pl.BlockSpec((H2, H2), lambda v: (0, 0)),
            pl.BlockSpec((8, 128), lambda v: (0, 0)),
        ],
        out_specs=[
            pl.BlockSpec(memory_space=pl.ANY),
            pl.BlockSpec((1, N, D_IN), lambda v: (v, 0, 0)),
            pl.BlockSpec((1, N, H2), lambda v: (v, 0, 0)),
            pl.BlockSpec((1, N, H2), lambda v: (v, 0, 0)),
            pl.BlockSpec((N, H2), lambda v: (0, 0)),
            pl.BlockSpec((N, H2), lambda v: (0, 0)),
        ],
        out_shape=[
            jax.ShapeDtypeStruct((V, N, N), f32),
            jax.ShapeDtypeStruct((V, N, D_IN), f32),
            jax.ShapeDtypeStruct((V, N, H2), f32),
            jax.ShapeDtypeStruct((V, N, H2), f32),
            jax.ShapeDtypeStruct((N, H2), f32),
            jax.ShapeDtypeStruct((N, H2), f32),
        ],
        scratch_shapes=[
            pltpu.VMEM((ND, TS, N), f32),
            pltpu.SemaphoreType.DMA((ND,)),
        ],
    )(feats, adjs, pm_W1, pm_b1r, pm_W2, pm_b2r,
      de_W1, de_b1r, de_W2, de_b2r, fg_W, fw)

    combined, q = pl.pallas_call(
        _combine_kernel,
        grid=(1,),
        in_specs=[
            pl.BlockSpec((V, N, H2), lambda i: (0, 0, 0)),
            pl.BlockSpec((V, N, H2), lambda i: (0, 0, 0)),
            pl.BlockSpec((N, H2), lambda i: (0, 0)),
            pl.BlockSpec((1, H2), lambda i: (0, 0)),
            pl.BlockSpec((8, 128), lambda i: (0, 0)),
            pl.BlockSpec((K, H2), lambda i: (0, 0)),
        ],
        out_specs=[
            pl.BlockSpec((N, H2), lambda i: (0, 0)),
            pl.BlockSpec((N, K), lambda i: (0, 0)),
        ],
        out_shape=[
            jax.ShapeDtypeStruct((N, H2), f32),
            jax.ShapeDtypeStruct((N, K), f32),
        ],
    )(m, g, gsum, fg_br, fw, centers)

    return (combined, combined_pr, q, xbar, adjbar)
